# baseline (speedup 1.0000x reference)
import functools

import jax
import jax.numpy as jnp
from jax import lax
from jax.experimental import pallas as pl
from jax.experimental.pallas import tpu as pltpu
from jax.experimental.pallas import tpu_sc as plsc

NEIGH = 10
EPS = 1e-5
BK = 2048
LANES = 128


def _batcher_pairs(n):
    pairs = []
    p = 1
    while p < n:
        k = p
        while k >= 1:
            for j in range(k % p, n - k, 2 * k):
                for i in range(0, min(k, n - j - k)):
                    if (i + j) // (2 * p) == (i + j + k) // (2 * p):
                        pairs.append((i + j, i + j + k))
            k //= 2
        p *= 2
    return pairs


def _prune(pairs, needed):
    needed = set(needed)
    kept = []
    for a, b in reversed(pairs):
        if a in needed or b in needed:
            kept.append((a, b))
            needed.add(a)
            needed.add(b)
    return list(reversed(kept))


_SORT16 = _prune(_batcher_pairs(16), range(NEIGH))
_CLEAN10 = [(0, 5), (1, 6), (2, 7), (3, 8), (4, 9), (1, 3), (0, 4), (2, 4),
            (1, 2), (3, 4), (5, 6), (7, 8), (5, 7), (6, 8), (6, 7), (5, 9),
            (7, 9), (6, 7), (8, 9)]


def _tc_body(q_ref, m_ref, cand2_ref, sum_out_ref,
             cand_ref, q2s_ref, qn_ref, sum_ref, sq_ref, qbar_ref,
             *, k_valid, grid):
    i = pl.program_id(0)
    q_rows = q_ref.shape[0]

    @pl.when(i == 0)
    def _init():
        for j in range(NEIGH):
            cand_ref[:, j * LANES:(j + 1) * LANES] = jnp.full(
                (q_rows, LANES), -jnp.inf, jnp.float32)
        q = q_ref[...]
        q2 = jnp.sum(q * q, axis=1, keepdims=True)
        q2s_ref[...] = q2
        qn_ref[...] = q * -2.0
        qbar_ref[...] = jnp.sum(q, axis=0, keepdims=True)
        sum_ref[0] = 0.0
        sq_ref[0] = jnp.sum(q2)

    m = m_ref[...]
    row = i * BK + jax.lax.broadcasted_iota(jnp.int32, (BK, 1), 0)
    m = jnp.where(row < k_valid, m, 0.0)
    dotn = jax.lax.dot_general(
        qn_ref[...], m, (((1,), (1,)), ((), ())),
        preferred_element_type=jnp.float32,
        precision=jax.lax.Precision.DEFAULT)
    m2 = jnp.sum(m * m, axis=1)
    col = i * BK + jax.lax.broadcasted_iota(jnp.int32, (1, BK), 1)
    m2m = jnp.where(col < k_valid, m2[None, :], -jnp.inf)
    d2m = m2m + dotn
    nvalid = jnp.minimum(k_valid - i * BK, BK).astype(jnp.float32)
    s_m2 = jnp.sum(m2)
    mbar = jnp.sum(m, axis=0, keepdims=True)
    qm = jnp.sum(qbar_ref[...] * mbar)
    sum_ref[0] += nvalid * sq_ref[0] + q_rows * s_m2 - 2.0 * qm

    s = [d2m[:, c * LANES:(c + 1) * LANES] for c in range(BK // LANES)]
    for a, b in _SORT16:
        hi = jnp.maximum(s[a], s[b])
        lo = jnp.minimum(s[a], s[b])
        s[a], s[b] = hi, lo
    slots = [cand_ref[:, j * LANES:(j + 1) * LANES] for j in range(NEIGH)]
    t = [jnp.maximum(slots[j], s[NEIGH - 1 - j]) for j in range(NEIGH)]
    for a, b in _CLEAN10:
        hi = jnp.maximum(t[a], t[b])
        lo = jnp.minimum(t[a], t[b])
        t[a], t[b] = hi, lo
    for j in range(NEIGH):
        cand_ref[:, j * LANES:(j + 1) * LANES] = t[j]

    @pl.when(i == grid - 1)
    def _final():
        cur = t
        width = LANES
        for _ in range(2):
            half = width // 2
            a = [x[:, :half] for x in cur]
            b = [x[:, half:] for x in cur]
            nxt = [jnp.maximum(a[j], b[NEIGH - 1 - j]) for j in range(NEIGH)]
            for p_, q_ in _CLEAN10:
                hi = jnp.maximum(nxt[p_], nxt[q_])
                lo = jnp.minimum(nxt[p_], nxt[q_])
                nxt[p_], nxt[q_] = hi, lo
            cur = nxt
            width = half
        q2 = q2s_ref[...]
        for j in range(NEIGH):
            cand2_ref[:, j * width:(j + 1) * width] = cur[j] + q2
        sum_out_ref[...] = jnp.full((1, LANES), sum_ref[0], jnp.float32)


def _tc_stage(query, memory):
    q_rows, d = query.shape
    k_valid = memory.shape[0]
    grid = pl.cdiv(k_valid, BK)
    w = NEIGH * LANES

    w2 = w // 4
    body = functools.partial(_tc_body, k_valid=k_valid, grid=grid)
    cand2, sumv = pl.pallas_call(
        body,
        grid=(grid,),
        in_specs=[
            pl.BlockSpec((q_rows, d), lambda i: (0, 0)),
            pl.BlockSpec((BK, d), lambda i: (i, 0)),
        ],
        out_specs=[
            pl.BlockSpec((q_rows, w2), lambda i: (0, 0)),
            pl.BlockSpec((1, LANES), lambda i: (0, 0)),
        ],
        out_shape=[
            jax.ShapeDtypeStruct((q_rows, w2), jnp.float32),
            jax.ShapeDtypeStruct((1, LANES), jnp.float32),
        ],
        scratch_shapes=[
            pltpu.VMEM((q_rows, w), jnp.float32),
            pltpu.VMEM((q_rows, 1), jnp.float32),
            pltpu.VMEM((q_rows, d), jnp.float32),
            pltpu.SMEM((1,), jnp.float32),
            pltpu.SMEM((1,), jnp.float32),
            pltpu.VMEM((1, LANES), jnp.float32),
        ],
        compiler_params=pltpu.CompilerParams(
            dimension_semantics=("arbitrary",)),
    )(query, memory)
    return cand2, sumv


def _make_sc_select(q_rows, w, inv_qk):
    info = plsc.get_sparse_core_info()
    nc, ns, nl = info.num_cores, info.num_subcores, info.num_lanes
    nw = nc * ns
    rows_per = q_rows // nw
    vecs_per_row = w // nl
    mesh = plsc.VectorSubcoreMesh(core_axis_name="c", subcore_axis_name="s")

    @functools.partial(
        pl.kernel, mesh=mesh,
        out_type=jax.ShapeDtypeStruct((q_rows,), jnp.float32),
        compiler_params=pltpu.CompilerParams(needs_layout_passes=False),
        scratch_types=[
            pltpu.VMEM((rows_per, w), jnp.float32),
            pltpu.VMEM((nl,), jnp.float32),
            pltpu.VMEM((rows_per,), jnp.float32),
        ],
    )
    def sc_select(cand_hbm, sum_hbm, out_hbm, cand_v, sum_v, srow_v):
        wid = lax.axis_index("s") * nc + lax.axis_index("c")
        base = wid * rows_per
        pltpu.sync_copy(cand_hbm.at[pl.ds(base, rows_per)], cand_v)
        pltpu.sync_copy(sum_hbm.at[0, pl.ds(0, nl)], sum_v)
        mean_vec = sum_v[...] * inv_qk

        lanes = lax.iota(jnp.int32, nl)
        ninf = jnp.full((nl,), -jnp.inf, jnp.float32)
        for g in range(rows_per // nl):
            rid = lanes + g * nl

            def cand_body(c, slots, rid=rid):
                for u in range(4):
                    cid = lax.broadcast_in_dim(c * 4 + u, (nl,), ())
                    cur = plsc.load_gather(cand_v, [rid, cid])
                    new = []
                    for sj in slots:
                        hi = jnp.maximum(sj, cur)
                        cur = jnp.minimum(sj, cur)
                        new.append(hi)
                    slots = tuple(new)
                return slots

            slots = lax.fori_loop(0, w // 4, cand_body, (ninf,) * NEIGH)
            s = jnp.zeros((nl,), jnp.float32)
            for sj in slots:
                d2 = jnp.maximum(sj, 0.0)
                s = s + EPS / (d2 / mean_vec + EPS)
            x = s + EPS
            i32 = plsc.bitcast(x, jnp.int32)
            y = plsc.bitcast(jnp.int32(0x5F3759DF) - (i32 >> 1), jnp.float32)
            for _ in range(3):
                y = y * (1.5 - 0.5 * x * y * y)
            srow_v[pl.ds(g * nl, nl)] = y
        pltpu.sync_copy(srow_v, out_hbm.at[pl.ds(base, rows_per)])

    return sc_select


def kernel(query, memory):
    q_rows = query.shape[0]
    k_valid = memory.shape[0]
    w2 = NEIGH * LANES // 4
    cand2, sumv = _tc_stage(query, memory)
    sc_select = _make_sc_select(q_rows, w2, 1.0 / (q_rows * k_valid))
    out = sc_select(cand2, sumv)
    return out


# speedup vs baseline: 1.0359x; 1.0359x over previous
import functools

import jax
import jax.numpy as jnp
from jax import lax
from jax.experimental import pallas as pl
from jax.experimental.pallas import tpu as pltpu
from jax.experimental.pallas import tpu_sc as plsc

NEIGH = 10
EPS = 1e-5
BK = 2048
LANES = 128


_SORT16 = [(0, 1), (2, 3), (4, 5), (6, 7), (8, 9), (10, 11), (12, 13),
           (14, 15), (0, 2), (1, 3), (4, 6), (5, 7), (8, 10), (9, 11),
           (12, 14), (13, 15), (0, 4), (1, 5), (2, 6), (3, 7), (8, 12),
           (9, 13), (10, 14), (11, 15), (0, 8), (1, 9), (2, 10), (3, 11),
           (4, 12), (5, 13), (6, 14), (7, 15), (5, 10), (6, 9), (3, 12),
           (13, 14), (7, 11), (1, 2), (4, 8), (1, 4), (7, 13), (2, 8),
           (5, 6), (9, 10), (2, 4), (3, 8), (7, 12), (6, 8), (10, 12),
           (3, 5), (7, 9), (3, 4), (5, 6), (7, 8), (9, 10), (6, 7), (8, 9)]
_CLEAN10 = [(0, 5), (1, 6), (2, 7), (3, 8), (4, 9), (1, 3), (0, 4), (2, 4),
            (1, 2), (3, 4), (5, 6), (7, 8), (5, 7), (6, 8), (6, 7), (5, 9),
            (7, 9), (6, 7), (8, 9)]


def _tc_body(q_ref, m_ref, cand2_ref, sum_out_ref,
             cand_ref, q2s_ref, qn_ref, sum_ref, sq_ref, qbar_ref,
             *, k_valid, grid):
    i = pl.program_id(0)
    q_rows = q_ref.shape[0]

    @pl.when(i == 0)
    def _init():
        for j in range(NEIGH):
            cand_ref[:, j * LANES:(j + 1) * LANES] = jnp.full(
                (q_rows, LANES), -jnp.inf, jnp.float32)
        q = q_ref[...]
        q2 = jnp.sum(q * q, axis=1, keepdims=True)
        q2s_ref[...] = q2
        qn_ref[...] = q * -2.0
        qbar_ref[...] = jnp.sum(q, axis=0, keepdims=True)
        sum_ref[0] = 0.0
        sq_ref[0] = jnp.sum(q2)

    m = m_ref[...]
    row = i * BK + jax.lax.broadcasted_iota(jnp.int32, (BK, 1), 0)
    m = jnp.where(row < k_valid, m, 0.0)
    dotn = jax.lax.dot_general(
        qn_ref[...], m, (((1,), (1,)), ((), ())),
        preferred_element_type=jnp.float32,
        precision=jax.lax.Precision.DEFAULT)
    m2 = jnp.sum(m * m, axis=1)
    col = i * BK + jax.lax.broadcasted_iota(jnp.int32, (1, BK), 1)
    m2m = jnp.where(col < k_valid, m2[None, :], -jnp.inf)
    d2m = m2m + dotn
    nvalid = jnp.minimum(k_valid - i * BK, BK).astype(jnp.float32)
    s_m2 = jnp.sum(m2)
    mbar = jnp.sum(m, axis=0, keepdims=True)
    qm = jnp.sum(qbar_ref[...] * mbar)
    sum_ref[0] += nvalid * sq_ref[0] + q_rows * s_m2 - 2.0 * qm

    s = [d2m[:, c * LANES:(c + 1) * LANES] for c in range(BK // LANES)]
    for a, b in _SORT16:
        hi = jnp.maximum(s[a], s[b])
        lo = jnp.minimum(s[a], s[b])
        s[a], s[b] = hi, lo
    slots = [cand_ref[:, j * LANES:(j + 1) * LANES] for j in range(NEIGH)]
    t = [jnp.maximum(slots[j], s[NEIGH - 1 - j]) for j in range(NEIGH)]
    for a, b in _CLEAN10:
        hi = jnp.maximum(t[a], t[b])
        lo = jnp.minimum(t[a], t[b])
        t[a], t[b] = hi, lo
    for j in range(NEIGH):
        cand_ref[:, j * LANES:(j + 1) * LANES] = t[j]

    @pl.when(i == grid - 1)
    def _final():
        cur = t
        width = LANES
        for _ in range(2):
            half = width // 2
            a = [x[:, :half] for x in cur]
            b = [x[:, half:] for x in cur]
            nxt = [jnp.maximum(a[j], b[NEIGH - 1 - j]) for j in range(NEIGH)]
            for p_, q_ in _CLEAN10:
                hi = jnp.maximum(nxt[p_], nxt[q_])
                lo = jnp.minimum(nxt[p_], nxt[q_])
                nxt[p_], nxt[q_] = hi, lo
            cur = nxt
            width = half
        q2 = q2s_ref[...]
        for j in range(NEIGH):
            cand2_ref[:, j * width:(j + 1) * width] = cur[j] + q2
        sum_out_ref[...] = jnp.full((1, LANES), sum_ref[0], jnp.float32)


def _tc_stage(query, memory):
    q_rows, d = query.shape
    k_valid = memory.shape[0]
    grid = pl.cdiv(k_valid, BK)
    w = NEIGH * LANES

    w2 = w // 4
    body = functools.partial(_tc_body, k_valid=k_valid, grid=grid)
    cand2, sumv = pl.pallas_call(
        body,
        grid=(grid,),
        in_specs=[
            pl.BlockSpec((q_rows, d), lambda i: (0, 0)),
            pl.BlockSpec((BK, d), lambda i: (i, 0)),
        ],
        out_specs=[
            pl.BlockSpec((q_rows, w2), lambda i: (0, 0)),
            pl.BlockSpec((1, LANES), lambda i: (0, 0)),
        ],
        out_shape=[
            jax.ShapeDtypeStruct((q_rows, w2), jnp.float32),
            jax.ShapeDtypeStruct((1, LANES), jnp.float32),
        ],
        scratch_shapes=[
            pltpu.VMEM((q_rows, w), jnp.float32),
            pltpu.VMEM((q_rows, 1), jnp.float32),
            pltpu.VMEM((q_rows, d), jnp.float32),
            pltpu.SMEM((1,), jnp.float32),
            pltpu.SMEM((1,), jnp.float32),
            pltpu.VMEM((1, LANES), jnp.float32),
        ],
        compiler_params=pltpu.CompilerParams(
            dimension_semantics=("arbitrary",)),
    )(query, memory)
    return cand2, sumv


def _make_sc_select(q_rows, w, inv_qk):
    info = plsc.get_sparse_core_info()
    nc, ns, nl = info.num_cores, info.num_subcores, info.num_lanes
    nw = nc * ns
    rows_per = q_rows // nw
    vecs_per_row = w // nl
    mesh = plsc.VectorSubcoreMesh(core_axis_name="c", subcore_axis_name="s")

    @functools.partial(
        pl.kernel, mesh=mesh,
        out_type=jax.ShapeDtypeStruct((q_rows,), jnp.float32),
        compiler_params=pltpu.CompilerParams(needs_layout_passes=False),
        scratch_types=[
            pltpu.VMEM((rows_per, w), jnp.float32),
            pltpu.VMEM((nl,), jnp.float32),
            pltpu.VMEM((rows_per,), jnp.float32),
        ],
    )
    def sc_select(cand_hbm, sum_hbm, out_hbm, cand_v, sum_v, srow_v):
        wid = lax.axis_index("s") * nc + lax.axis_index("c")
        base = wid * rows_per
        pltpu.sync_copy(cand_hbm.at[pl.ds(base, rows_per)], cand_v)
        pltpu.sync_copy(sum_hbm.at[0, pl.ds(0, nl)], sum_v)
        mean_vec = sum_v[...] * inv_qk

        lanes = lax.iota(jnp.int32, nl)
        ninf = jnp.full((nl,), -jnp.inf, jnp.float32)
        for g in range(rows_per // nl):
            rid = lanes + g * nl

            def cand_body(c, slots, rid=rid):
                for u in range(4):
                    cid = lax.broadcast_in_dim(c * 4 + u, (nl,), ())
                    cur = plsc.load_gather(cand_v, [rid, cid])
                    new = []
                    for sj in slots:
                        hi = jnp.maximum(sj, cur)
                        cur = jnp.minimum(sj, cur)
                        new.append(hi)
                    slots = tuple(new)
                return slots

            slots = lax.fori_loop(0, w // 4, cand_body, (ninf,) * NEIGH)
            s = jnp.zeros((nl,), jnp.float32)
            for sj in slots:
                d2 = jnp.maximum(sj, 0.0)
                s = s + EPS / (d2 / mean_vec + EPS)
            x = s + EPS
            i32 = plsc.bitcast(x, jnp.int32)
            y = plsc.bitcast(jnp.int32(0x5F3759DF) - (i32 >> 1), jnp.float32)
            for _ in range(3):
                y = y * (1.5 - 0.5 * x * y * y)
            srow_v[pl.ds(g * nl, nl)] = y
        pltpu.sync_copy(srow_v, out_hbm.at[pl.ds(base, rows_per)])

    return sc_select


def kernel(query, memory):
    q_rows = query.shape[0]
    k_valid = memory.shape[0]
    w2 = NEIGH * LANES // 4
    cand2, sumv = _tc_stage(query, memory)
    sc_select = _make_sc_select(q_rows, w2, 1.0 / (q_rows * k_valid))
    out = sc_select(cand2, sumv)
    return out
